# baseline (speedup 1.0000x reference)
import functools

import jax
import jax.numpy as jnp
from jax import lax
from jax.experimental import pallas as pl
from jax.experimental.pallas import tpu as pltpu
from jax.experimental.pallas import tpu_sc as plsc


def _make_proj_manual(V, D, NH, chunk, nbuf):
    ns = V // chunk

    def body(t_hbm, w_ref, b_ref, o_hbm, bin_ref, bout_ref, sin, sout):
        for i in range(nbuf):
            pltpu.make_async_copy(
                t_hbm.at[pl.ds(i * chunk, chunk)], bin_ref.at[i], sin.at[i]
            ).start()

        def step(s, carry):
            slot = lax.rem(s, nbuf)
            pltpu.make_async_copy(
                t_hbm.at[pl.ds(s * chunk, chunk)], bin_ref.at[slot], sin.at[slot]
            ).wait()

            @pl.when(s >= nbuf)
            def _():
                pltpu.make_async_copy(
                    bout_ref.at[slot],
                    o_hbm.at[pl.ds((s - nbuf) * chunk, chunk)],
                    sout.at[slot],
                ).wait()

            bout_ref[slot] = jnp.dot(
                bin_ref[slot], w_ref[...], preferred_element_type=jnp.float32
            ) + b_ref[...]
            pltpu.make_async_copy(
                bout_ref.at[slot], o_hbm.at[pl.ds(s * chunk, chunk)], sout.at[slot]
            ).start()

            @pl.when(s + nbuf < ns)
            def _():
                pltpu.make_async_copy(
                    t_hbm.at[pl.ds((s + nbuf) * chunk, chunk)],
                    bin_ref.at[slot],
                    sin.at[slot],
                ).start()

            return carry

        lax.fori_loop(0, ns, step, 0)

        def drain(k, carry):
            s = ns - nbuf + k
            slot = lax.rem(s, nbuf)
            pltpu.make_async_copy(
                bout_ref.at[slot], o_hbm.at[pl.ds(s * chunk, chunk)], sout.at[slot]
            ).wait()
            return carry

        lax.fori_loop(0, nbuf, drain, 0)

    return pl.pallas_call(
        body,
        in_specs=[
            pl.BlockSpec(memory_space=pltpu.MemorySpace.HBM),
            pl.BlockSpec(memory_space=pltpu.MemorySpace.VMEM),
            pl.BlockSpec(memory_space=pltpu.MemorySpace.VMEM),
        ],
        out_specs=pl.BlockSpec(memory_space=pltpu.MemorySpace.HBM),
        out_shape=jax.ShapeDtypeStruct((V, NH), jnp.float32),
        scratch_shapes=[
            pltpu.VMEM((nbuf, chunk, D), jnp.float32),
            pltpu.VMEM((nbuf, chunk, NH), jnp.float32),
            pltpu.SemaphoreType.DMA((nbuf,)),
            pltpu.SemaphoreType.DMA((nbuf,)),
        ],
    )


def _make_gather(B, L, NH, n_workers, chunk):
    N = B * L
    b_rows = B // n_workers
    b_per_w = N // n_workers
    n_super = b_per_w // (2 * chunk)
    mesh = plsc.VectorSubcoreMesh(core_axis_name="c", subcore_axis_name="s")

    @functools.partial(
        pl.kernel,
        mesh=mesh,
        out_type=jax.ShapeDtypeStruct((N, NH), jnp.float32),
        scratch_types=[
            pltpu.VMEM((b_per_w,), jnp.int32),
            pltpu.VMEM((2, chunk, NH), jnp.float32),
            pltpu.SemaphoreType.DMA((2,)),
        ],
    )
    def gather_k(idx_hbm, p_hbm, out_hbm, idx_v, rows_v, sem):
        nc = lax.axis_size("c")
        wid = lax.axis_index("s") * nc + lax.axis_index("c")
        base = wid * b_per_w
        pltpu.sync_copy(idx_hbm.at[pl.ds(base, b_per_w)], idx_v)

        def start_gather(c, slot):
            pltpu.async_copy(
                p_hbm.at[idx_v.at[pl.ds(c * chunk, chunk)]],
                rows_v.at[slot],
                sem.at[slot],
            )

        def wait_gather(slot):
            pltpu.make_async_copy(
                p_hbm.at[pl.ds(0, chunk)], rows_v.at[slot], sem.at[slot]
            ).wait()

        start_gather(0, 0)

        def sbody(g, carry):
            c0 = 2 * g
            start_gather(c0 + 1, 1)
            wait_gather(0)
            pltpu.sync_copy(
                rows_v.at[0], out_hbm.at[pl.ds(base + c0 * chunk, chunk)]
            )

            @pl.when(g + 1 < n_super)
            def _():
                start_gather(c0 + 2, 0)

            wait_gather(1)
            pltpu.sync_copy(
                rows_v.at[1], out_hbm.at[pl.ds(base + (c0 + 1) * chunk, chunk)]
            )
            return carry

        lax.fori_loop(0, n_super, sbody, 0)

    return gather_k


def kernel(encoder_word, table, W, b):
    B, L = encoder_word.shape
    V, D = table.shape
    H = W.shape[0]
    N = B * L
    NH = 128

    Wp = jnp.zeros((D, NH), jnp.float32).at[:, :H].set(W.T)
    bp = jnp.zeros((1, NH), jnp.float32)

    proj = _make_proj_manual(V, D, NH, chunk=2000, nbuf=12)
    P_pad = proj(table, Wp, bp)

    info = plsc.get_sparse_core_info()
    n_workers = info.num_cores * info.num_subcores
    gather_k = _make_gather(B, L, NH, n_workers, chunk=256)
    idx = encoder_word.reshape(N).astype(jnp.int32)
    out_pad = gather_k(idx, P_pad)
    return out_pad[:, :H].reshape(B, L, H) + b


# speedup vs baseline: 1.0928x; 1.0928x over previous
import functools

import jax
import jax.numpy as jnp
from jax import lax
from jax.experimental import pallas as pl
from jax.experimental.pallas import tpu as pltpu
from jax.experimental.pallas import tpu_sc as plsc


def _make_proj_manual(V, D, NH, chunk, nbuf):
    ns = V // chunk

    def body(t_hbm, w_ref, b_ref, o_hbm, bin_ref, bout_ref, sin, sout):
        for i in range(nbuf):
            pltpu.make_async_copy(
                t_hbm.at[pl.ds(i * chunk, chunk)], bin_ref.at[i], sin.at[i]
            ).start()

        def step(s, carry):
            slot = lax.rem(s, nbuf)
            pltpu.make_async_copy(
                t_hbm.at[pl.ds(s * chunk, chunk)], bin_ref.at[slot], sin.at[slot]
            ).wait()

            @pl.when(s >= nbuf)
            def _():
                pltpu.make_async_copy(
                    bout_ref.at[slot],
                    o_hbm.at[pl.ds((s - nbuf) * chunk, chunk)],
                    sout.at[slot],
                ).wait()

            bout_ref[slot] = jnp.dot(
                bin_ref[slot], w_ref[...], preferred_element_type=jnp.float32
            ) + b_ref[...]
            pltpu.make_async_copy(
                bout_ref.at[slot], o_hbm.at[pl.ds(s * chunk, chunk)], sout.at[slot]
            ).start()

            @pl.when(s + nbuf < ns)
            def _():
                pltpu.make_async_copy(
                    t_hbm.at[pl.ds((s + nbuf) * chunk, chunk)],
                    bin_ref.at[slot],
                    sin.at[slot],
                ).start()

            return carry

        lax.fori_loop(0, ns, step, 0)

        def drain(k, carry):
            s = ns - nbuf + k
            slot = lax.rem(s, nbuf)
            pltpu.make_async_copy(
                bout_ref.at[slot], o_hbm.at[pl.ds(s * chunk, chunk)], sout.at[slot]
            ).wait()
            return carry

        lax.fori_loop(0, nbuf, drain, 0)

    return pl.pallas_call(
        body,
        in_specs=[
            pl.BlockSpec(memory_space=pltpu.MemorySpace.HBM),
            pl.BlockSpec(memory_space=pltpu.MemorySpace.VMEM),
            pl.BlockSpec(memory_space=pltpu.MemorySpace.VMEM),
        ],
        out_specs=pl.BlockSpec(memory_space=pltpu.MemorySpace.HBM),
        out_shape=jax.ShapeDtypeStruct((V, NH), jnp.float32),
        scratch_shapes=[
            pltpu.VMEM((nbuf, chunk, D), jnp.float32),
            pltpu.VMEM((nbuf, chunk, NH), jnp.float32),
            pltpu.SemaphoreType.DMA((nbuf,)),
            pltpu.SemaphoreType.DMA((nbuf,)),
        ],
    )


def _make_gather(B, L, NH, n_workers, chunk):
    N = B * L
    b_rows = B // n_workers
    b_per_w = N // n_workers
    n_super = b_per_w // (2 * chunk)
    mesh = plsc.VectorSubcoreMesh(core_axis_name="c", subcore_axis_name="s")

    @functools.partial(
        pl.kernel,
        mesh=mesh,
        out_type=jax.ShapeDtypeStruct((N, NH), jnp.float32),
        scratch_types=[
            pltpu.VMEM((b_per_w,), jnp.int32),
            pltpu.VMEM((2, chunk, NH), jnp.float32),
            pltpu.SemaphoreType.DMA((2,)),
        ],
    )
    def gather_k(idx_hbm, p_hbm, out_hbm, idx_v, rows_v, sem):
        nc = lax.axis_size("c")
        wid = lax.axis_index("s") * nc + lax.axis_index("c")
        base = wid * b_per_w
        pltpu.sync_copy(idx_hbm.at[pl.ds(base, b_per_w)], idx_v)

        def start_gather(c, slot):
            pltpu.async_copy(
                p_hbm.at[idx_v.at[pl.ds(c * chunk, chunk)]],
                rows_v.at[slot],
                sem.at[slot],
            )

        def wait_gather(slot):
            pltpu.make_async_copy(
                p_hbm.at[pl.ds(0, chunk)], rows_v.at[slot], sem.at[slot]
            ).wait()

        start_gather(0, 0)

        def sbody(g, carry):
            c0 = 2 * g
            start_gather(c0 + 1, 1)
            wait_gather(0)
            pltpu.sync_copy(
                rows_v.at[0], out_hbm.at[pl.ds(base + c0 * chunk, chunk)]
            )

            @pl.when(g + 1 < n_super)
            def _():
                start_gather(c0 + 2, 0)

            wait_gather(1)
            pltpu.sync_copy(
                rows_v.at[1], out_hbm.at[pl.ds(base + (c0 + 1) * chunk, chunk)]
            )
            return carry

        lax.fori_loop(0, n_super, sbody, 0)

    return gather_k


def kernel(encoder_word, table, W, b):
    B, L = encoder_word.shape
    V, D = table.shape
    H = W.shape[0]
    N = B * L
    NH = 128

    Wp = jnp.zeros((D, NH), jnp.float32).at[:, :H].set(W.T)
    bp = jnp.zeros((1, NH), jnp.float32).at[:, :H].set(b)

    proj = _make_proj_manual(V, D, NH, chunk=2000, nbuf=12)
    P_pad = proj(table, Wp, bp)

    info = plsc.get_sparse_core_info()
    n_workers = info.num_cores * info.num_subcores
    gather_k = _make_gather(B, L, NH, n_workers, chunk=320)
    idx = encoder_word.reshape(N).astype(jnp.int32)
    out_pad = gather_k(idx, P_pad)
    return out_pad[:, :H].reshape(B, L, H)
